# baseline (speedup 1.0000x reference)
import numpy as np
import jax
import jax.numpy as jnp
from jax.experimental import pallas as pl
from jax.experimental.pallas import tpu as pltpu

_P = 24
_R = 3
_NBINS = _P + 2
_H = 384
_W = 384

_BC = 128
_SCOLS = 640


def _offsets():
    offs = []
    for i in range(_P):
        theta = 2.0 * np.pi * i / _P
        rp = float(np.round(-_R * np.sin(theta), 8))
        cp = float(np.round(_R * np.cos(theta), 8))
        minr = int(np.floor(rp))
        minc = int(np.floor(cp))
        tr = rp - minr
        tc = cp - minc
        offs.append((minr, minc, tr, tc))
    return offs

_OFFS = _offsets()


def _lbp_kernel(img_ref, out_ref, rs_ref):

    @pl.when(pl.program_id(0) == 0)
    def _init():
        rs_ref[...] = jnp.zeros((8, _H, _SCOLS), jnp.float32)

    x = jnp.clip(img_ref[0], 0.0, 1.0)
    x = jnp.floor(x * 255.0)
    gray = jnp.round(0.299 * x[0] + 0.587 * x[1] + 0.114 * x[2])

    for dr in range(-3, 5):
        if dr > 0:
            sh = jnp.concatenate(
                [gray[dr:, :], jnp.zeros((dr, _W), jnp.float32)], axis=0)
        elif dr < 0:
            sh = jnp.concatenate(
                [jnp.zeros((-dr, _W), jnp.float32), gray[:dr, :]], axis=0)
        else:
            sh = gray
        rs_ref[dr + 3, :, _BC:_BC + _W] = sh

    def tap(minr, minc):
        c0 = _BC + minc
        return rs_ref[minr + 3, :, c0:c0 + _W]

    word = jnp.zeros((_H, _W), jnp.int32)
    for i, (minr, minc, tr, tc) in enumerate(_OFFS):
        terms = []
        w00 = (1.0 - tr) * (1.0 - tc)
        w10 = tr * (1.0 - tc)
        w01 = (1.0 - tr) * tc
        w11 = tr * tc
        if w00 != 0.0:
            terms.append(w00 * tap(minr, minc) if w00 != 1.0 else tap(minr, minc))
        if w10 != 0.0:
            terms.append(w10 * tap(minr + 1, minc))
        if w01 != 0.0:
            terms.append(w01 * tap(minr, minc + 1))
        if w11 != 0.0:
            terms.append(w11 * tap(minr + 1, minc + 1))
        neigh = terms[0]
        for t in terms[1:]:
            neigh = neigh + t
        bit = neigh >= gray
        word = word | jnp.where(bit, jnp.int32(1 << i), jnp.int32(0))

    ones_i = jax.lax.population_count(word)
    nxt = ((word >> 1) | (word << 23)) & jnp.int32(0xFFFFFF)
    trans = jax.lax.population_count(word ^ nxt)
    code = jnp.where(trans <= 2, ones_i, jnp.int32(_P + 1))
    counts = []
    for k in range(_NBINS):
        sel = jnp.where(code == k, 1.0, 0.0)
        part = jnp.sum(sel, axis=0, keepdims=True)
        counts.append(jnp.sum(part))

    inv_n = 1.0 / float(_H * _W)
    hvals = [c * inv_n for c in counts]
    total = hvals[0]
    for v in hvals[1:]:
        total = total + v
    denom = total + 1e-7
    hvec = jnp.concatenate(
        [jnp.broadcast_to((v / denom)[None, None], (1, 1)) for v in hvals],
        axis=1)
    out_ref[...] = hvec[None]


def kernel(images):
    B = images.shape[0]
    out = pl.pallas_call(
        _lbp_kernel,
        grid=(B,),
        in_specs=[pl.BlockSpec((1, 3, _H, _W), lambda b: (b, 0, 0, 0))],
        out_specs=pl.BlockSpec((1, 1, _NBINS), lambda b: (b, 0, 0)),
        out_shape=jax.ShapeDtypeStruct((B, 1, _NBINS), jnp.float32),
        scratch_shapes=[pltpu.VMEM((8, _H, _SCOLS), jnp.float32)],
    )(images)
    return out.reshape(B, _NBINS)


# speedup vs baseline: 1.7348x; 1.7348x over previous
import numpy as np
import jax
import jax.numpy as jnp
from jax.experimental import pallas as pl
from jax.experimental.pallas import tpu as pltpu

_P = 24
_R = 3
_NBINS = _P + 2
_H = 384
_W = 384

_BR = 8
_CROWS = 400


def _neighbor_offsets():
    offs = []
    for i in range(_P):
        theta = 2.0 * np.pi * i / _P
        rp = float(np.round(-_R * np.sin(theta), 8))
        cp = float(np.round(_R * np.cos(theta), 8))
        minr = int(np.floor(rp))
        minc = int(np.floor(cp))
        tr = rp - minr
        tc = cp - minc
        offs.append((minr, minc, tr, tc))
    return offs

_OFFS = _neighbor_offsets()


def _lbp_kernel(img_ref, out_ref, cs_ref):

    cs_ref[:, _BR - 3:_BR, :] = jnp.zeros((7, 3, _W), jnp.float32)
    cs_ref[:, _BR + _H:_BR + _H + 4, :] = jnp.zeros((7, 4, _W), jnp.float32)

    x = jnp.clip(img_ref[0], 0.0, 1.0)
    x = jnp.floor(x * 255.0)
    gray = jnp.round(0.299 * x[0] + 0.587 * x[1] + 0.114 * x[2])

    for j in range(-3, 4):
        if j > 0:
            sh = jnp.concatenate(
                [gray[:, j:], jnp.zeros((_H, j), jnp.float32)], axis=1)
        elif j < 0:
            sh = jnp.concatenate(
                [jnp.zeros((_H, -j), jnp.float32), gray[:, :j]], axis=1)
        else:
            sh = gray
        cs_ref[j + 3, _BR:_BR + _H, :] = sh

    def tap(minr, minc):
        r0 = _BR + minr
        return cs_ref[minc + 3, r0:r0 + _H, :]

    word = jnp.zeros((_H, _W), jnp.int32)
    for i, (minr, minc, tr, tc) in enumerate(_OFFS):
        terms = []
        w00 = (1.0 - tr) * (1.0 - tc)
        w10 = tr * (1.0 - tc)
        w01 = (1.0 - tr) * tc
        w11 = tr * tc
        if w00 != 0.0:
            terms.append(w00 * tap(minr, minc) if w00 != 1.0 else tap(minr, minc))
        if w10 != 0.0:
            terms.append(w10 * tap(minr + 1, minc))
        if w01 != 0.0:
            terms.append(w01 * tap(minr, minc + 1))
        if w11 != 0.0:
            terms.append(w11 * tap(minr + 1, minc + 1))
        neigh = terms[0]
        for t in terms[1:]:
            neigh = neigh + t
        bit = neigh >= gray
        word = word | jnp.where(bit, jnp.int32(1 << i), jnp.int32(0))

    ones_i = jax.lax.population_count(word)
    nxt = ((word >> 1) | (word << 23)) & jnp.int32(0xFFFFFF)
    trans = jax.lax.population_count(word ^ nxt)
    code = jnp.where(trans <= 2, ones_i, jnp.int32(_P + 1))
    counts = []
    for k in range(_NBINS):
        sel = jnp.where(code == k, 1.0, 0.0)
        part = jnp.sum(sel, axis=0, keepdims=True)
        counts.append(jnp.sum(part))

    inv_n = 1.0 / float(_H * _W)
    hvals = [c * inv_n for c in counts]
    total = hvals[0]
    for v in hvals[1:]:
        total = total + v
    denom = total + 1e-7
    hvec = jnp.concatenate(
        [jnp.broadcast_to((v / denom)[None, None], (1, 1)) for v in hvals],
        axis=1)
    out_ref[...] = hvec[None]


def kernel(images):
    B = images.shape[0]
    out = pl.pallas_call(
        _lbp_kernel,
        grid=(B,),
        in_specs=[pl.BlockSpec((1, 3, _H, _W), lambda b: (b, 0, 0, 0))],
        out_specs=pl.BlockSpec((1, 1, _NBINS), lambda b: (b, 0, 0)),
        out_shape=jax.ShapeDtypeStruct((B, 1, _NBINS), jnp.float32),
        scratch_shapes=[pltpu.VMEM((7, _CROWS, _W), jnp.float32)],
        compiler_params=pltpu.CompilerParams(
            dimension_semantics=("parallel",)),
    )(images)
    return out.reshape(B, _NBINS)
